# Initial kernel scaffold; baseline (speedup 1.0000x reference)
import jax
import jax.numpy as jnp
from jax.experimental import pallas as pl


def kernel(input_ids, table, gamma, beta):
    raise NotImplementedError("write your pallas kernel here")


# speedup vs baseline: 8.5871x; 8.5871x over previous
import functools

import jax
import jax.numpy as jnp
from jax import lax
from jax.experimental import pallas as pl
from jax.experimental.pallas import tpu as pltpu
from jax.experimental.pallas import tpu_sc as plsc

EPS = 1e-12


def _ln_table_kernel(t_ref, g_ref, b_ref, o_ref):
    t = t_ref[...]
    m = jnp.mean(t, axis=1, keepdims=True)
    c = t - m
    v = jnp.mean(c * c, axis=1, keepdims=True)
    o_ref[...] = c * lax.rsqrt(v + EPS) * g_ref[...] + b_ref[...]


def _normalize_table(table, gamma, beta):
    V, D = table.shape
    return pl.pallas_call(
        _ln_table_kernel,
        out_shape=jax.ShapeDtypeStruct((V, D), jnp.float32),
    )(table, gamma.reshape(1, D), beta.reshape(1, D))


@functools.lru_cache(maxsize=None)
def _make_gather(V, D, N):
    info = plsc.get_sparse_core_info()
    NC, NS = info.num_cores, info.num_subcores
    NW = NC * NS
    CHUNK = 128
    assert N % (NW * CHUNK) == 0
    n_chunks = N // (NW * CHUNK)
    NBUF = 2

    mesh = plsc.VectorSubcoreMesh(core_axis_name="c", subcore_axis_name="s")

    @functools.partial(
        pl.kernel,
        mesh=mesh,
        out_type=jax.ShapeDtypeStruct((N, D), jnp.float32),
        scratch_types=[
            pltpu.VMEM((n_chunks, CHUNK), jnp.int32),
            pltpu.VMEM((NBUF, CHUNK, D), jnp.float32),
            pltpu.SemaphoreType.DMA,
            pltpu.SemaphoreType.DMA,
        ],
    )
    def gather(idx_hbm, tab_hbm, out_hbm, idx_v, rows_v, gsem, ssem):
        wid = lax.axis_index("s") * NC + lax.axis_index("c")
        base = wid * (n_chunks * CHUNK)
        pltpu.sync_copy(idx_hbm.at[wid], idx_v)

        def fire(j, buf):
            return pltpu.async_copy(tab_hbm.at[idx_v.at[j]], rows_v.at[buf], gsem)

        fire(0, 0)

        def body(j, _):
            buf = lax.rem(j, NBUF)
            nxt = lax.rem(j + 1, NBUF)

            @pl.when(j >= 1)
            def _():
                pltpu.make_async_copy(
                    rows_v.at[nxt],
                    out_hbm.at[pl.ds(base, CHUNK)],
                    ssem,
                ).wait()

            @pl.when(j + 1 < n_chunks)
            def _():
                fire(j + 1, nxt)

            pltpu.make_async_copy(
                tab_hbm.at[idx_v.at[j]], rows_v.at[buf], gsem
            ).wait()
            pltpu.async_copy(
                rows_v.at[buf],
                out_hbm.at[pl.ds(base + j * CHUNK, CHUNK)],
                ssem,
            )
            return 0

        lax.fori_loop(0, n_chunks, body, 0)
        pltpu.make_async_copy(
            rows_v.at[lax.rem(n_chunks - 1, NBUF)],
            out_hbm.at[pl.ds(base, CHUNK)],
            ssem,
        ).wait()

    return gather


def kernel(input_ids, table, gamma, beta):
    B, L = input_ids.shape
    V, D = table.shape
    N = B * L
    normed = _normalize_table(table, gamma, beta)
    info = plsc.get_sparse_core_info()
    NW = info.num_cores * info.num_subcores
    idx = input_ids.reshape(NW, N // (NW * 128), 128).astype(jnp.int32)
    out = _make_gather(V, D, N)(idx, normed)
    return out.reshape(B, L, D)


# speedup vs baseline: 8.6738x; 1.0101x over previous
import functools

import jax
import jax.numpy as jnp
from jax import lax
from jax.experimental import pallas as pl
from jax.experimental.pallas import tpu as pltpu
from jax.experimental.pallas import tpu_sc as plsc

EPS = 1e-12


def _ln_table_kernel(t_ref, g_ref, b_ref, o_ref):
    t = t_ref[...]
    m = jnp.mean(t, axis=1, keepdims=True)
    c = t - m
    v = jnp.mean(c * c, axis=1, keepdims=True)
    o_ref[...] = c * lax.rsqrt(v + EPS) * g_ref[...] + b_ref[...]


def _normalize_table(table, gamma, beta):
    V, D = table.shape
    return pl.pallas_call(
        _ln_table_kernel,
        out_shape=jax.ShapeDtypeStruct((V, D), jnp.float32),
    )(table, gamma.reshape(1, D), beta.reshape(1, D))


@functools.lru_cache(maxsize=None)
def _make_gather(V, D, N):
    info = plsc.get_sparse_core_info()
    NC, NS = info.num_cores, info.num_subcores
    NW = NC * NS
    CHUNK = 128
    assert N % (NW * CHUNK) == 0
    n_chunks = N // (NW * CHUNK)
    NBUF = 4
    GA = NBUF - 1

    mesh = plsc.VectorSubcoreMesh(core_axis_name="c", subcore_axis_name="s")

    @functools.partial(
        pl.kernel,
        mesh=mesh,
        out_type=jax.ShapeDtypeStruct((N, D), jnp.float32),
        scratch_types=[
            pltpu.VMEM((n_chunks, CHUNK), jnp.int32),
            pltpu.VMEM((NBUF, CHUNK, D), jnp.float32),
            pltpu.SemaphoreType.DMA,
            pltpu.SemaphoreType.DMA,
        ],
    )
    def gather(idx_hbm, tab_hbm, out_hbm, idx_v, rows_v, gsem, ssem):
        wid = lax.axis_index("s") * NC + lax.axis_index("c")
        base = wid * (n_chunks * CHUNK)
        pltpu.sync_copy(idx_hbm.at[wid], idx_v)

        def fire(j, buf):
            return pltpu.async_copy(tab_hbm.at[idx_v.at[j]], rows_v.at[buf], gsem)

        def drain_one_scatter():
            pltpu.make_async_copy(
                rows_v.at[0],
                out_hbm.at[pl.ds(base, CHUNK)],
                ssem,
            ).wait()

        for j0 in range(GA):
            fire(j0, j0)

        def body(j, _):
            buf = lax.rem(j, NBUF)

            @pl.when(j + GA < n_chunks)
            def _():
                @pl.when(j >= 1)
                def _():
                    drain_one_scatter()

                fire(j + GA, lax.rem(j + GA, NBUF))

            pltpu.make_async_copy(
                tab_hbm.at[idx_v.at[j]], rows_v.at[buf], gsem
            ).wait()
            pltpu.async_copy(
                rows_v.at[buf],
                out_hbm.at[pl.ds(base + j * CHUNK, CHUNK)],
                ssem,
            )
            return 0

        lax.fori_loop(0, n_chunks, body, 0)
        for _ in range(NBUF):
            drain_one_scatter()

    return gather


def kernel(input_ids, table, gamma, beta):
    B, L = input_ids.shape
    V, D = table.shape
    N = B * L
    normed = _normalize_table(table, gamma, beta)
    info = plsc.get_sparse_core_info()
    NW = info.num_cores * info.num_subcores
    idx = input_ids.reshape(NW, N // (NW * 128), 128).astype(jnp.int32)
    out = _make_gather(V, D, N)(idx, normed)
    return out.reshape(B, L, D)
